# baseline (speedup 1.0000x reference)
import math

import jax
import jax.numpy as jnp
from jax.experimental import pallas as pl
from jax.experimental.pallas import tpu as pltpu

HIDDEN = 256
LAYERS = 4
HEAD = HIDDEN // LAYERS
K = 10
B = 16
NNODE = 128
N = B * NNODE

_BINOM = [math.comb(K, j) / (2.0 ** K) for j in range(K + 1)]


def _poly_coeffs(j):
    c = [0] * (K + 1)
    for p in range(j + 1):
        for q in range(K - j + 1):
            c[p + q] += (-1) ** p * math.comb(j, p) * math.comb(K - j, q)
    return c


_T = [[float(_poly_coeffs(j)[i]) for j in range(K + 1)] for i in range(K + 1)]


def _mm(a, b):
    return jax.lax.dot_general(
        a, b, (((1,), (0,)), ((), ())),
        precision=jax.lax.Precision.DEFAULT,
        preferred_element_type=jnp.float32)


def _mmb(a, b):
    return jax.lax.dot_general(
        a.astype(jnp.bfloat16), b.astype(jnp.bfloat16),
        (((1,), (0,)), ((), ())),
        preferred_element_type=jnp.float32)


_BPP = 8


def _bern_kernel(adj_ref, x_ref, coe_ref,
                 w0_ref, b0_ref, w1_ref, b1_ref, w2_ref, b2_ref,
                 w3_ref, b3_ref, wout_ref, bout_ref, out_ref):
    temp = jnp.maximum(coe_ref[0], 0.0)
    c = []
    for i in range(K + 1):
        ci = None
        for j in range(K + 1):
            w = _T[i][j] * _BINOM[j]
            if w == 0.0:
                continue
            term = temp[j] * w
            ci = term if ci is None else ci + term
        c.append(ci if ci is not None else 0.0)

    ws = [w0_ref, w1_ref, w2_ref, w3_ref]
    bs = [b0_ref, b1_ref, b2_ref, b3_ref]

    Ms, M4s = [], []
    for p in range(_BPP):
        A = adj_ref[p]
        mask = (A > 0).astype(jnp.float32)
        deg = jnp.sum(mask, axis=0)
        dinv = jnp.where(deg > 0,
                         1.0 / jnp.sqrt(jnp.maximum(deg, 1.0)),
                         0.0)
        Ms.append(dinv[:, None] * mask.T * dinv[None, :])
    M2s = [_mmb(Ms[p], Ms[p]) for p in range(_BPP)]
    M4s = [_mmb(M2s[p], M2s[p]) for p in range(_BPP)]

    x0s = [x_ref[p] for p in range(_BPP)]
    caches = [[x0s[p]] for p in range(_BPP)]
    for layer in range(LAYERS):
        xps = []
        for p in range(_BPP):
            xin = (caches[p][0] if layer == 0
                   else jnp.concatenate(caches[p], axis=1))
            xps.append(_mm(xin, ws[layer][...]) + bs[layer][0][None, :])
        x1s = [_mmb(Ms[p], xps[p]) for p in range(_BPP)]
        x2s = [_mmb(Ms[p], x1s[p]) for p in range(_BPP)]
        x3s = [_mmb(Ms[p], x2s[p]) for p in range(_BPP)]
        ss = []
        for p in range(_BPP):
            pw = [xps[p], x1s[p], x2s[p], x3s[p]]
            c2x = c[8] * pw[0] + c[9] * pw[1] + c[10] * pw[2]
            c1x = c[4] * pw[0] + c[5] * pw[1] + c[6] * pw[2] + c[7] * pw[3]
            c0x = c[0] * pw[0] + c[1] * pw[1] + c[2] * pw[2] + c[3] * pw[3]
            ss.append((c1x, c0x, c2x))
        vs = [_mmb(M4s[p], ss[p][2]) + ss[p][0] for p in range(_BPP)]
        vs = [_mmb(M4s[p], vs[p]) + ss[p][1] for p in range(_BPP)]
        for p in range(_BPP):
            caches[p].append(jnp.maximum(vs[p], 0.0))

    for p in range(_BPP):
        bern = jnp.concatenate(caches[p][1:], axis=1) + x0s[p]
        out_ref[p] = _mm(bern, wout_ref[...]) + bout_ref[0][None, :]


def kernel(adj, input, coe, W0, b0, W1, b1, W2, b2, W3, b3, Wout, bout):
    coe2 = coe.reshape(1, K + 1)
    biases = [b.reshape(1, -1) for b in (b0, b1, b2, b3, bout)]

    def fixed(arr):
        nd = arr.ndim
        return pl.BlockSpec(arr.shape, lambda b: (0,) * nd)

    in_specs = [
        pl.BlockSpec((_BPP, NNODE, NNODE), lambda b: (b, 0, 0)),
        pl.BlockSpec((_BPP, NNODE, HIDDEN), lambda b: (b, 0, 0)),
        fixed(coe2),
        fixed(W0), fixed(biases[0]),
        fixed(W1), fixed(biases[1]),
        fixed(W2), fixed(biases[2]),
        fixed(W3), fixed(biases[3]),
        fixed(Wout), fixed(biases[4]),
    ]
    out = pl.pallas_call(
        _bern_kernel,
        grid=(B // _BPP,),
        in_specs=in_specs,
        out_specs=pl.BlockSpec((_BPP, NNODE, HIDDEN), lambda b: (b, 0, 0)),
        out_shape=jax.ShapeDtypeStruct((B, NNODE, HIDDEN), jnp.float32),
        compiler_params=pltpu.CompilerParams(
            dimension_semantics=("parallel",)),
    )(adj, input, coe2, W0, biases[0], W1, biases[1], W2, biases[2],
      W3, biases[3], Wout, biases[4])
    return out


# speedup vs baseline: 1.1172x; 1.1172x over previous
import math

import jax
import jax.numpy as jnp
from jax.experimental import pallas as pl
from jax.experimental.pallas import tpu as pltpu

HIDDEN = 256
LAYERS = 4
HEAD = HIDDEN // LAYERS
K = 10
B = 16
NNODE = 128
N = B * NNODE

_BINOM = [math.comb(K, j) / (2.0 ** K) for j in range(K + 1)]


def _poly_coeffs(j):
    c = [0] * (K + 1)
    for p in range(j + 1):
        for q in range(K - j + 1):
            c[p + q] += (-1) ** p * math.comb(j, p) * math.comb(K - j, q)
    return c


_T = [[float(_poly_coeffs(j)[i]) for j in range(K + 1)] for i in range(K + 1)]


def _mm(a, b):
    return jax.lax.dot_general(
        a, b, (((1,), (0,)), ((), ())),
        precision=jax.lax.Precision.DEFAULT,
        preferred_element_type=jnp.float32)


def _mmT(a, b):
    return jax.lax.dot_general(
        a, b, (((0,), (0,)), ((), ())),
        precision=jax.lax.Precision.DEFAULT,
        preferred_element_type=jnp.float32)


_BPP = 16


def _bern_kernel(adj_ref, x_ref, coe_ref,
                 w0_ref, b0_ref, w1_ref, b1_ref, w2_ref, b2_ref,
                 w3_ref, b3_ref, wout_ref, bout_ref, out_ref):
    temp = jnp.maximum(coe_ref[0], 0.0)
    c = []
    for i in range(K + 1):
        ci = None
        for j in range(K + 1):
            w = _T[i][j] * _BINOM[j]
            if w == 0.0:
                continue
            term = temp[j] * w
            ci = term if ci is None else ci + term
        c.append(ci if ci is not None else 0.0)

    ws = [w0_ref, w1_ref, w2_ref, w3_ref]
    bs = [b0_ref, b1_ref, b2_ref, b3_ref]

    MTs = []
    for p in range(_BPP):
        A = adj_ref[p]
        mask = (A > 0).astype(jnp.float32)
        deg = jnp.sum(mask, axis=0)
        dinv = 1.0 / jnp.sqrt(jnp.maximum(deg, 1.0))
        MTs.append(dinv[:, None] * mask * dinv[None, :])
    M2Ts = [_mm(MTs[p], MTs[p]) for p in range(_BPP)]
    M4Ts = [_mm(M2Ts[p], M2Ts[p]) for p in range(_BPP)]

    x0s = [x_ref[p] for p in range(_BPP)]
    caches = [[x0s[p]] for p in range(_BPP)]
    for layer in range(LAYERS):
        xps = []
        for p in range(_BPP):
            xin = (caches[p][0] if layer == 0
                   else jnp.concatenate(caches[p], axis=1))
            xps.append(_mm(xin, ws[layer][...]) + bs[layer][0][None, :])
        x1s = [_mmT(MTs[p], xps[p]) for p in range(_BPP)]
        x2s = [_mmT(MTs[p], x1s[p]) for p in range(_BPP)]
        x3s = [_mmT(MTs[p], x2s[p]) for p in range(_BPP)]
        ss = []
        for p in range(_BPP):
            pw = [xps[p], x1s[p], x2s[p], x3s[p]]
            c2x = c[8] * pw[0] + c[9] * pw[1] + c[10] * pw[2]
            c1x = c[4] * pw[0] + c[5] * pw[1] + c[6] * pw[2] + c[7] * pw[3]
            c0x = c[0] * pw[0] + c[1] * pw[1] + c[2] * pw[2] + c[3] * pw[3]
            ss.append((c1x, c0x, c2x))
        vs = [_mmT(M4Ts[p], ss[p][2]) + ss[p][0] for p in range(_BPP)]
        vs = [_mmT(M4Ts[p], vs[p]) + ss[p][1] for p in range(_BPP)]
        for p in range(_BPP):
            caches[p].append(jnp.maximum(vs[p], 0.0))

    for p in range(_BPP):
        bern = jnp.concatenate(caches[p][1:], axis=1) + x0s[p]
        out_ref[p] = _mm(bern, wout_ref[...]) + bout_ref[0][None, :]


def kernel(adj, input, coe, W0, b0, W1, b1, W2, b2, W3, b3, Wout, bout):
    coe2 = coe.reshape(1, K + 1)
    biases = [b.reshape(1, -1) for b in (b0, b1, b2, b3, bout)]

    def fixed(arr):
        nd = arr.ndim
        return pl.BlockSpec(arr.shape, lambda b: (0,) * nd)

    in_specs = [
        pl.BlockSpec((_BPP, NNODE, NNODE), lambda b: (b, 0, 0)),
        pl.BlockSpec((_BPP, NNODE, HIDDEN), lambda b: (b, 0, 0)),
        fixed(coe2),
        fixed(W0), fixed(biases[0]),
        fixed(W1), fixed(biases[1]),
        fixed(W2), fixed(biases[2]),
        fixed(W3), fixed(biases[3]),
        fixed(Wout), fixed(biases[4]),
    ]
    out = pl.pallas_call(
        _bern_kernel,
        grid=(B // _BPP,),
        in_specs=in_specs,
        out_specs=pl.BlockSpec((_BPP, NNODE, HIDDEN), lambda b: (b, 0, 0)),
        out_shape=jax.ShapeDtypeStruct((B, NNODE, HIDDEN), jnp.float32),
        compiler_params=pltpu.CompilerParams(
            dimension_semantics=("parallel",)),
    )(adj, input, coe2, W0, biases[0], W1, biases[1], W2, biases[2],
      W3, biases[3], Wout, biases[4])
    return out
